# baseline (speedup 1.0000x reference)
import jax
import jax.numpy as jnp
from jax import lax
from jax.experimental import pallas as pl
from jax.experimental.pallas import tpu as pltpu
from jax.experimental.pallas import tpu_sc as plsc

_W = 128
_LB = 8
_NW = 32
_OP = 136


def _gather_t(weight_pairs, idx_t, n_l, n_b, d):
    n_pairs = weight_pairs.shape[0]
    lb_tiles = idx_t.shape[0] // _LB
    n_bb = n_b // _W
    per_tile = lb_tiles * n_bb // _NW
    mesh = plsc.VectorSubcoreMesh(core_axis_name="core", subcore_axis_name="subcore")

    @pl.kernel(
        out_type=jax.ShapeDtypeStruct((n_l, d, n_b), jnp.float32),
        mesh=mesh,
        scratch_types=[
            pltpu.VMEM((_LB, _W), jnp.int32),
            pltpu.VMEM((_LB, _W), jnp.int32),
            pltpu.VMEM((2, _W, 2 * d), jnp.float32),
            pltpu.VMEM((d, _OP), jnp.float32),
            pltpu.SemaphoreType.DMA,
            pltpu.SemaphoreType.DMA,
        ],
        compiler_params=pltpu.CompilerParams(
            use_tc_tiling_on_sc=True, needs_layout_passes=False
        ),
    )
    def kern(x_hbm, i_hbm, o_hbm, ir_v, ip_v, g_v, o_v, gsem, osem):
        wid = lax.axis_index("subcore") * 2 + lax.axis_index("core")
        iotas = [lax.iota(jnp.int32, 16) + c * 16 for c in range(4)]

        @pl.loop(0, per_tile)
        def _(s):
            sw = wid * per_tile + s
            lb = sw // n_bb
            bb = sw % n_bb
            pltpu.sync_copy(
                i_hbm.at[pl.ds(lb * _LB, _LB), pl.ds(bb * _W, _W)], ir_v
            )
            for lr in range(_LB):
                for t in range(_W // 16):
                    v = ir_v[lr, pl.ds(t * 16, 16)]
                    ip_v[lr, pl.ds(t * 16, 16)] = jnp.minimum(v >> 1, n_pairs - 1)
            copies = [
                pltpu.async_copy(x_hbm.at[ip_v.at[lr]], g_v.at[lr % 2], gsem)
                for lr in range(2)
            ]
            for lr in range(_LB):
                l = lb * _LB + lr
                copies[lr].wait()

                @pl.when(l < n_l)
                def _():
                    g = g_v.at[lr % 2]

                    @pl.loop(0, _W // 16)
                    def _(t):
                        hv = ir_v[lr, pl.ds(t * 16, 16)] & 1
                        for lane in range(16):
                            off = hv[lane] * d
                            b = t * 16 + lane
                            cols = jnp.full((16,), b, jnp.int32)
                            for c in range(4):
                                v = g[b, pl.ds(off + c * 16, 16)]
                                plsc.store_scatter(o_v, [iotas[c], cols], v)

                    pltpu.sync_copy(
                        o_v.at[:, pl.ds(0, _W)],
                        o_hbm.at[l, :, pl.ds(bb * _W, _W)],
                    )

                if lr + 2 < _LB:
                    copies.append(
                        pltpu.async_copy(
                            x_hbm.at[ip_v.at[lr + 2]], g_v.at[lr % 2], gsem
                        )
                    )

    return kern(weight_pairs, idx_t)


def kernel(indices, weight):
    b, l = indices.shape
    d = weight.shape[1]
    idx_t = indices.transpose(1, 0).astype(jnp.int32)
    lb_pad = (l + _LB - 1) // _LB * _LB
    idx_t = jnp.pad(idx_t, ((0, lb_pad - l), (0, 0)))
    weight_pairs = weight.reshape(-1, 2 * d)
    out_t = _gather_t(weight_pairs, idx_t, l, b, d)
    return out_t.transpose(2, 0, 1)


# speedup vs baseline: 3.8193x; 3.8193x over previous
import jax
import jax.numpy as jnp
from jax.experimental import pallas as pl
from jax.experimental.pallas import tpu as pltpu
from jax.experimental.pallas import tpu_sc as plsc

_WINDOW = 128
_J = 4


def _gather_flat(weight, idx_2d):
    n_win, _ = idx_2d.shape
    n = n_win * _WINDOW
    d = weight.shape[1]
    mesh = plsc.VectorSubcoreMesh(core_axis_name="core", subcore_axis_name="subcore")

    @pl.kernel(
        out_type=jax.ShapeDtypeStruct((n, d), weight.dtype),
        mesh=mesh,
        scratch_types=[pltpu.SemaphoreType.DMA],
        compiler_params=pltpu.CompilerParams(use_tc_tiling_on_sc=False),
    )
    def kern(x_hbm, i_hbm, o_hbm, sem):
        def body(i_vmem, o_vmem):
            copies = [
                pltpu.async_copy(
                    x_hbm.at[i_vmem.at[j]],
                    o_vmem.at[pl.ds(j * _WINDOW, _WINDOW)],
                    sem,
                )
                for j in range(_J)
            ]
            for c in copies:
                c.wait()

        pltpu.emit_pipeline(
            body,
            grid=(n_win // _J,),
            in_specs=[pl.BlockSpec((_J, _WINDOW), index_map=lambda i: (i, 0))],
            out_specs=[pl.BlockSpec((_J * _WINDOW, d), index_map=lambda i: (i, 0))],
            core_axis_name=("core", "subcore"),
            dimension_semantics=(pltpu.PARALLEL,),
        )(i_hbm, o_hbm)

    return kern(weight, idx_2d)


def kernel(indices, weight):
    b, l = indices.shape
    idx_2d = indices.reshape(-1, _WINDOW).astype(jnp.int32)
    out = _gather_flat(weight, idx_2d)
    return out.reshape(b, l, weight.shape[1])
